# baseline (speedup 1.0000x reference)
import functools

import jax
import jax.numpy as jnp
from jax import lax
from jax.experimental import pallas as pl
from jax.experimental.pallas import tpu as pltpu

N_CLUSTERS = 1000
D_EMB = 1024
D_MODEL = 1024
R = 16
TOPK = 50
BETA = 0.2
TAU = 0.01
SCALING = 2.0


def _merge_kernel(q_ref, corpus_ref, w_base_ref, a_hbm, b_hbm, out_ref,
                  idx_smem, val_smem, a_buf, b_buf, a_sem, b_sem):
    f32 = jnp.float32

    q = q_ref[...]
    qn = q * jax.lax.rsqrt(jnp.sum(q * q) + 1e-30)
    qnorm = jnp.sqrt(jnp.sum(q * q))
    qn = q / (qnorm + 1e-9)
    corpus = corpus_ref[...]
    dots = jax.lax.dot_general(
        qn, corpus, (((1,), (1,)), ((), ())),
        preferred_element_type=f32,
        precision=jax.lax.Precision.HIGHEST)
    ones_row = jnp.ones((1, D_EMB), f32)
    sq = jax.lax.dot_general(
        ones_row, corpus * corpus, (((1,), (1,)), ((), ())),
        preferred_element_type=f32,
        precision=jax.lax.Precision.HIGHEST)
    cnorm = jnp.sqrt(sq)
    sim = dots / (cnorm + 1e-9) / (BETA * BETA)

    m = jnp.max(sim)
    e = jnp.exp(sim - m)
    probs = e / jnp.sum(e)
    probs = jnp.where(probs >= TAU, probs, 0.0)

    iota = jax.lax.broadcasted_iota(jnp.int32, (1, N_CLUSTERS), 1)

    def topk_body(k, carry):
        p, total, nnz = carry
        v = jnp.max(p)
        i = jnp.min(jnp.where(p == v, iota, N_CLUSTERS))
        i = jnp.minimum(i, N_CLUSTERS - 1)
        idx_smem[k] = i
        val_smem[k] = v
        p = jnp.where(iota == i, -1.0, p)
        total = total + v
        nnz = nnz + jnp.where(v > 0.0, 1, 0)
        return p, total, nnz

    _, total, nnz = lax.fori_loop(
        0, TOPK, topk_body, (probs, jnp.float32(0.0), jnp.int32(0)))
    inv_total = SCALING / (total + 1e-9)

    out_ref[...] = w_base_ref[...]

    def start_fetch(k, slot):
        i = idx_smem[k]
        pltpu.make_async_copy(a_hbm.at[i], a_buf.at[slot], a_sem.at[slot]).start()
        pltpu.make_async_copy(b_hbm.at[i], b_buf.at[slot], b_sem.at[slot]).start()

    @pl.when(nnz > 0)
    def _():
        start_fetch(0, 0)

    def merge_body(k, _):
        slot = lax.rem(k, 2)

        @pl.when(k + 1 < nnz)
        def _():
            start_fetch(k + 1, lax.rem(k + 1, 2))

        pltpu.make_async_copy(a_hbm.at[0], a_buf.at[slot], a_sem.at[slot]).wait()
        pltpu.make_async_copy(b_hbm.at[0], b_buf.at[slot], b_sem.at[slot]).wait()
        w = val_smem[k] * inv_total
        a = a_buf[slot] * w
        b = b_buf[slot]
        out_ref[...] += jax.lax.dot_general(
            b, a, (((1,), (0,)), ((), ())),
            preferred_element_type=f32,
            precision=jax.lax.Precision.HIGHEST)
        return 0

    lax.fori_loop(0, nnz, merge_body, 0)


@jax.jit
def kernel(q, corpus, A_all, B_all, W_base):
    return pl.pallas_call(
        _merge_kernel,
        out_shape=jax.ShapeDtypeStruct((D_MODEL, D_MODEL), jnp.float32),
        in_specs=[
            pl.BlockSpec(memory_space=pltpu.MemorySpace.VMEM),
            pl.BlockSpec(memory_space=pltpu.MemorySpace.VMEM),
            pl.BlockSpec(memory_space=pltpu.MemorySpace.VMEM),
            pl.BlockSpec(memory_space=pltpu.MemorySpace.HBM),
            pl.BlockSpec(memory_space=pltpu.MemorySpace.HBM),
        ],
        out_specs=pl.BlockSpec(memory_space=pltpu.MemorySpace.VMEM),
        scratch_shapes=[
            pltpu.SMEM((TOPK + 1,), jnp.int32),
            pltpu.SMEM((TOPK + 1,), jnp.float32),
            pltpu.VMEM((2, R, D_MODEL), jnp.float32),
            pltpu.VMEM((2, D_MODEL, R), jnp.float32),
            pltpu.SemaphoreType.DMA((2,)),
            pltpu.SemaphoreType.DMA((2,)),
        ],
    )(q, corpus, W_base, A_all, B_all)


# speedup vs baseline: 1.0533x; 1.0533x over previous
import functools

import jax
import jax.numpy as jnp
from jax import lax
from jax.experimental import pallas as pl
from jax.experimental.pallas import tpu as pltpu

N_CLUSTERS = 1000
D_EMB = 1024
D_MODEL = 1024
R = 16
TOPK = 50
BETA = 0.2
TAU = 0.01
SCALING = 2.0


def _merge_kernel(q_ref, corpus_ref, w_base_ref, a_hbm, b_hbm, out_ref,
                  idx_smem, val_smem, a_buf, b_buf, a_sem, b_sem):
    f32 = jnp.float32

    q = q_ref[...]
    qn = q * jax.lax.rsqrt(jnp.sum(q * q) + 1e-30)
    qnorm = jnp.sqrt(jnp.sum(q * q))
    qn = q / (qnorm + 1e-9)
    corpus = corpus_ref[...]
    dots = jax.lax.dot_general(
        qn, corpus, (((1,), (1,)), ((), ())),
        preferred_element_type=f32,
        precision=jax.lax.Precision.HIGHEST)
    ones_row = jnp.ones((1, D_EMB), f32)
    sq = jax.lax.dot_general(
        ones_row, corpus * corpus, (((1,), (1,)), ((), ())),
        preferred_element_type=f32,
        precision=jax.lax.Precision.HIGHEST)
    cnorm = jnp.sqrt(sq)
    sim = dots / (cnorm + 1e-9) / (BETA * BETA)

    m = jnp.max(sim)
    e = jnp.exp(sim - m)
    probs = e / jnp.sum(e)
    probs = jnp.where(probs >= TAU, probs, 0.0)

    iota = jax.lax.broadcasted_iota(jnp.int32, (1, N_CLUSTERS), 1)
    count = jnp.sum(jnp.where(probs > 0.0, 1, 0))
    nnz = jnp.minimum(count, TOPK)

    def topk_body(k, carry):
        p, total = carry
        v = jnp.max(p)
        i = jnp.min(jnp.where(p == v, iota, N_CLUSTERS))
        i = jnp.minimum(i, N_CLUSTERS - 1)
        idx_smem[k] = i
        val_smem[k] = v
        p = jnp.where(iota == i, -1.0, p)
        return p, total + v

    _, total = lax.fori_loop(
        0, nnz, topk_body, (probs, jnp.float32(0.0)))
    inv_total = SCALING / (total + 1e-9)

    out_ref[...] = w_base_ref[...]

    def start_fetch(k, slot):
        i = idx_smem[k]
        pltpu.make_async_copy(a_hbm.at[i], a_buf.at[slot], a_sem.at[slot]).start()
        pltpu.make_async_copy(b_hbm.at[i], b_buf.at[slot], b_sem.at[slot]).start()

    @pl.when(nnz > 0)
    def _():
        start_fetch(0, 0)

    def merge_body(k, _):
        slot = lax.rem(k, 2)

        @pl.when(k + 1 < nnz)
        def _():
            start_fetch(k + 1, lax.rem(k + 1, 2))

        pltpu.make_async_copy(a_hbm.at[0], a_buf.at[slot], a_sem.at[slot]).wait()
        pltpu.make_async_copy(b_hbm.at[0], b_buf.at[slot], b_sem.at[slot]).wait()
        w = val_smem[k] * inv_total
        a = a_buf[slot] * w
        b = b_buf[slot]
        out_ref[...] += jax.lax.dot_general(
            b, a, (((1,), (0,)), ((), ())),
            preferred_element_type=f32,
            precision=jax.lax.Precision.HIGHEST)
        return 0

    lax.fori_loop(0, nnz, merge_body, 0)


@jax.jit
def kernel(q, corpus, A_all, B_all, W_base):
    return pl.pallas_call(
        _merge_kernel,
        out_shape=jax.ShapeDtypeStruct((D_MODEL, D_MODEL), jnp.float32),
        in_specs=[
            pl.BlockSpec(memory_space=pltpu.MemorySpace.VMEM),
            pl.BlockSpec(memory_space=pltpu.MemorySpace.VMEM),
            pl.BlockSpec(memory_space=pltpu.MemorySpace.VMEM),
            pl.BlockSpec(memory_space=pltpu.MemorySpace.HBM),
            pl.BlockSpec(memory_space=pltpu.MemorySpace.HBM),
        ],
        out_specs=pl.BlockSpec(memory_space=pltpu.MemorySpace.VMEM),
        scratch_shapes=[
            pltpu.SMEM((TOPK + 1,), jnp.int32),
            pltpu.SMEM((TOPK + 1,), jnp.float32),
            pltpu.VMEM((2, R, D_MODEL), jnp.float32),
            pltpu.VMEM((2, D_MODEL, R), jnp.float32),
            pltpu.SemaphoreType.DMA((2,)),
            pltpu.SemaphoreType.DMA((2,)),
        ],
    )(q, corpus, W_base, A_all, B_all)


# speedup vs baseline: 14.4452x; 13.7149x over previous
import jax
import jax.numpy as jnp
from jax import lax
from jax.experimental import pallas as pl
from jax.experimental.pallas import tpu as pltpu

N_CLUSTERS = 1000
D_EMB = 1024
D_MODEL = 1024
R = 16
TOPK = 50
BETA = 0.2
TAU = 0.01
SCALING = 2.0


def _routing_kernel(q_ref, corpus_ref, idx_ref, w_ref):
    f32 = jnp.float32
    q = q_ref[...]
    qn = q / (jnp.sqrt(jnp.sum(q * q)) + 1e-9)
    corpus = corpus_ref[...]
    dots = lax.dot_general(
        qn, corpus, (((1,), (1,)), ((), ())),
        preferred_element_type=f32,
        precision=lax.Precision.HIGHEST)
    sq = lax.dot_general(
        jnp.ones((1, D_EMB), f32), corpus * corpus, (((1,), (1,)), ((), ())),
        preferred_element_type=f32,
        precision=lax.Precision.HIGHEST)
    sim = dots / (jnp.sqrt(sq) + 1e-9) / (BETA * BETA)

    m = jnp.max(sim)
    e = jnp.exp(sim - m)
    probs = e / jnp.sum(e)
    probs = jnp.where(probs >= TAU, probs, 0.0)

    iota = lax.broadcasted_iota(jnp.int32, (1, N_CLUSTERS), 1)
    count = jnp.sum(jnp.where(probs > 0.0, 1, 0))
    nnz = jnp.minimum(count, TOPK)

    def topk_body(k, carry):
        p, total = carry
        v = jnp.max(p)
        i = jnp.min(jnp.where(p == v, iota, N_CLUSTERS))
        i = jnp.minimum(i, N_CLUSTERS - 1)
        idx_ref[k] = i
        w_ref[k] = v
        return jnp.where(iota == i, -1.0, p), total + v

    _, total = lax.fori_loop(0, nnz, topk_body, (probs, jnp.float32(0.0)))

    safe = jnp.where(nnz > 0, idx_ref[jnp.maximum(nnz - 1, 0)], 0)

    def tail_body(k, _):
        idx_ref[k] = safe
        w_ref[k] = 0.0
        return 0

    lax.fori_loop(nnz, TOPK, tail_body, 0)
    w_ref[TOPK] = SCALING / (total + 1e-9)


def _merge_kernel(idx_sm, w_sm, a_ref, bt_ref, wb_ref, o_ref, acat, btcat):
    k = pl.program_id(0)
    w = w_sm[k] * w_sm[TOPK]
    acat[pl.ds(k * R, R), :] = a_ref[0] * w
    btcat[pl.ds(k * R, R), :] = bt_ref[0]

    @pl.when(k == TOPK - 1)
    def _():
        o_ref[...] = wb_ref[...] + lax.dot_general(
            btcat[...], acat[...], (((0,), (0,)), ((), ())),
            preferred_element_type=jnp.float32)


@jax.jit
def kernel(q, corpus, A_all, B_all, W_base):
    Bt = jnp.swapaxes(B_all, 1, 2)
    idx, wts = pl.pallas_call(
        _routing_kernel,
        out_shape=(
            jax.ShapeDtypeStruct((TOPK,), jnp.int32),
            jax.ShapeDtypeStruct((TOPK + 1,), jnp.float32),
        ),
        in_specs=[
            pl.BlockSpec(memory_space=pltpu.MemorySpace.VMEM),
            pl.BlockSpec(memory_space=pltpu.MemorySpace.VMEM),
        ],
        out_specs=(
            pl.BlockSpec(memory_space=pltpu.MemorySpace.SMEM),
            pl.BlockSpec(memory_space=pltpu.MemorySpace.SMEM),
        ),
    )(q, corpus)

    grid_spec = pltpu.PrefetchScalarGridSpec(
        num_scalar_prefetch=2,
        grid=(TOPK,),
        in_specs=[
            pl.BlockSpec((1, R, D_MODEL), lambda k, i, w: (i[k], 0, 0)),
            pl.BlockSpec((1, R, D_MODEL), lambda k, i, w: (i[k], 0, 0)),
            pl.BlockSpec((D_MODEL, D_MODEL), lambda k, i, w: (0, 0)),
        ],
        out_specs=pl.BlockSpec((D_MODEL, D_MODEL), lambda k, i, w: (0, 0)),
        scratch_shapes=[
            pltpu.VMEM((TOPK * R, D_MODEL), jnp.float32),
            pltpu.VMEM((TOPK * R, D_MODEL), jnp.float32),
        ],
    )
    return pl.pallas_call(
        _merge_kernel,
        out_shape=jax.ShapeDtypeStruct((D_MODEL, D_MODEL), jnp.float32),
        grid_spec=grid_spec,
    )(idx, wts, A_all, Bt, W_base)


# speedup vs baseline: 28.4083x; 1.9666x over previous
import jax
import jax.numpy as jnp
from jax import lax
from jax.experimental import pallas as pl
from jax.experimental.pallas import tpu as pltpu

N_CLUSTERS = 1000
D_EMB = 1024
D_MODEL = 1024
R = 16
TOPK = 50
BETA = 0.2
TAU = 0.01
SCALING = 2.0


def _fused_kernel(q_ref, corpus_ref, wb_ref, a_hbm, bt_hbm, o_ref,
                  idx_smem, val_smem, a_buf, bt_buf, a_sem, bt_sem):
    f32 = jnp.float32

    q = q_ref[...]
    qn = q / (jnp.sqrt(jnp.sum(q * q)) + 1e-9)
    corpus = corpus_ref[...]
    dots = lax.dot_general(
        qn, corpus, (((1,), (1,)), ((), ())),
        preferred_element_type=f32)
    sq = lax.dot_general(
        jnp.ones((1, D_EMB), f32), corpus * corpus, (((1,), (1,)), ((), ())),
        preferred_element_type=f32)
    sim = dots / (jnp.sqrt(sq) + 1e-9) / (BETA * BETA)

    m = jnp.max(sim)
    e = jnp.exp(sim - m)
    probs = e / jnp.sum(e)
    probs = jnp.where(probs >= TAU, probs, 0.0)

    iota = lax.broadcasted_iota(jnp.int32, (1, N_CLUSTERS), 1)
    count = jnp.sum(jnp.where(probs > 0.0, 1, 0))
    nnz = jnp.minimum(count, TOPK)

    def topk_body(k, carry):
        p, total = carry
        v = jnp.max(p)
        i = jnp.min(jnp.where(p == v, iota, N_CLUSTERS))
        i = jnp.minimum(i, N_CLUSTERS - 1)
        idx_smem[k] = i
        val_smem[k] = v
        return jnp.where(iota == i, -1.0, p), total + v

    _, total = lax.fori_loop(0, nnz, topk_body, (probs, jnp.float32(0.0)))
    inv_total = SCALING / (total + 1e-9)

    out = wb_ref[...]

    def start_fetch(k, slot):
        i = idx_smem[k]
        pltpu.make_async_copy(a_hbm.at[i], a_buf.at[slot], a_sem.at[slot]).start()
        pltpu.make_async_copy(bt_hbm.at[i], bt_buf.at[slot], bt_sem.at[slot]).start()

    @pl.when(nnz > 0)
    def _():
        start_fetch(0, 0)

    def merge_body(k, acc):
        slot = lax.rem(k, 2)

        @pl.when(k + 1 < nnz)
        def _():
            start_fetch(k + 1, lax.rem(k + 1, 2))

        pltpu.make_async_copy(a_hbm.at[0], a_buf.at[slot], a_sem.at[slot]).wait()
        pltpu.make_async_copy(bt_hbm.at[0], bt_buf.at[slot], bt_sem.at[slot]).wait()
        w = val_smem[k] * inv_total
        return acc + lax.dot_general(
            bt_buf[slot], a_buf[slot] * w, (((0,), (0,)), ((), ())),
            preferred_element_type=f32)

    o_ref[...] = lax.fori_loop(0, nnz, merge_body, out)


@jax.jit
def kernel(q, corpus, A_all, B_all, W_base):
    Bt = jnp.swapaxes(B_all, 1, 2)
    return pl.pallas_call(
        _fused_kernel,
        out_shape=jax.ShapeDtypeStruct((D_MODEL, D_MODEL), jnp.float32),
        in_specs=[
            pl.BlockSpec(memory_space=pltpu.MemorySpace.VMEM),
            pl.BlockSpec(memory_space=pltpu.MemorySpace.VMEM),
            pl.BlockSpec(memory_space=pltpu.MemorySpace.VMEM),
            pl.BlockSpec(memory_space=pltpu.MemorySpace.HBM),
            pl.BlockSpec(memory_space=pltpu.MemorySpace.HBM),
        ],
        out_specs=pl.BlockSpec(memory_space=pltpu.MemorySpace.VMEM),
        scratch_shapes=[
            pltpu.SMEM((TOPK + 1,), jnp.int32),
            pltpu.SMEM((TOPK + 1,), jnp.float32),
            pltpu.VMEM((2, R, D_MODEL), jnp.float32),
            pltpu.VMEM((2, R, D_MODEL), jnp.float32),
            pltpu.SemaphoreType.DMA((2,)),
            pltpu.SemaphoreType.DMA((2,)),
        ],
    )(q, corpus, W_base, A_all, Bt)


# speedup vs baseline: 29.3113x; 1.0318x over previous
import jax
import jax.numpy as jnp
from jax import lax
from jax.experimental import pallas as pl
from jax.experimental.pallas import tpu as pltpu

N_CLUSTERS = 1000
D_EMB = 1024
D_MODEL = 1024
R = 16
TOPK = 50
BETA = 0.2
TAU = 0.01
SCALING = 2.0

_CHUNKS = (256, 256, 256, 232)
_BAND = 256
_NBANDS = D_MODEL // _BAND


def _fused_kernel(q_ref, corpus_hbm, wb_hbm, a_hbm, bt_hbm, o_hbm,
                  idx_smem, val_smem, c_buf, w_buf, a_buf, bt_buf, o_stage,
                  c_sem, w_sem, a_sem, bt_sem, o_sem):
    f32 = jnp.float32

    lo = 0
    for ci, sz in enumerate(_CHUNKS):
        pltpu.make_async_copy(
            corpus_hbm.at[pl.ds(lo, sz)], c_buf.at[pl.ds(lo, sz)],
            c_sem.at[ci]).start()
        lo += sz
    pltpu.make_async_copy(wb_hbm, w_buf, w_sem).start()

    q = q_ref[...]
    qn = q / (jnp.sqrt(jnp.sum(q * q)) + 1e-9)
    ones_row = jnp.ones((1, D_EMB), f32)

    def chunk_sims(lo, sz, ci):
        pltpu.make_async_copy(
            corpus_hbm.at[pl.ds(0, sz)], c_buf.at[pl.ds(0, sz)],
            c_sem.at[ci]).wait()
        c = c_buf[pl.ds(lo, sz), :]
        dots = lax.dot_general(
            qn, c, (((1,), (1,)), ((), ())),
            preferred_element_type=f32)
        sq = lax.dot_general(
            ones_row, c * c, (((1,), (1,)), ((), ())),
            preferred_element_type=f32)
        return dots / (jnp.sqrt(sq) + 1e-9) / (BETA * BETA)

    pieces = []
    lo = 0
    for ci, sz in enumerate(_CHUNKS):
        pieces.append(chunk_sims(lo, sz, ci))
        lo += sz
    sim = jnp.concatenate(pieces, axis=1)

    m = jnp.max(sim)
    e = jnp.exp(sim - m)
    probs = e / jnp.sum(e)
    probs = jnp.where(probs >= TAU, probs, 0.0)

    iota = lax.broadcasted_iota(jnp.int32, (1, N_CLUSTERS), 1)
    count = jnp.sum(jnp.where(probs > 0.0, 1, 0))
    nnz = jnp.minimum(count, TOPK)

    def topk_body(k, carry):
        p, total = carry
        v = jnp.max(p)
        i = jnp.min(jnp.where(p == v, iota, N_CLUSTERS))
        i = jnp.minimum(i, N_CLUSTERS - 1)
        idx_smem[k] = i
        val_smem[k] = v
        return jnp.where(iota == i, -1.0, p), total + v

    _, total = lax.fori_loop(0, nnz, topk_body, (probs, jnp.float32(0.0)))
    inv_total = SCALING / (total + 1e-9)

    def fire_body(k, _):
        i = idx_smem[k]
        pltpu.make_async_copy(a_hbm.at[i], a_buf.at[k], a_sem).start()
        pltpu.make_async_copy(bt_hbm.at[i], bt_buf.at[k], bt_sem).start()
        return 0

    lax.fori_loop(0, nnz, fire_body, 0)

    def drain_body(k, _):
        pltpu.make_async_copy(a_hbm.at[0], a_buf.at[0], a_sem).wait()
        pltpu.make_async_copy(bt_hbm.at[0], bt_buf.at[0], bt_sem).wait()
        a_buf[k] = a_buf[k] * (val_smem[k] * inv_total)
        return 0

    lax.fori_loop(0, nnz, drain_body, 0)

    pltpu.make_async_copy(wb_hbm, w_buf, w_sem).wait()

    for b in range(_NBANDS):
        slot = b % 2
        if b >= 2:
            pltpu.make_async_copy(
                o_stage.at[0], o_hbm.at[pl.ds(0, _BAND)], o_sem.at[slot]).wait()

        def band_body(k, acc):
            return acc + lax.dot_general(
                bt_buf[k, :, pl.ds(b * _BAND, _BAND)], a_buf[k],
                (((0,), (0,)), ((), ())),
                preferred_element_type=f32)

        band = lax.fori_loop(
            0, nnz, band_body, w_buf[pl.ds(b * _BAND, _BAND), :])
        o_stage[slot] = band
        pltpu.make_async_copy(
            o_stage.at[slot], o_hbm.at[pl.ds(b * _BAND, _BAND)],
            o_sem.at[slot]).start()

    for slot in range(2):
        pltpu.make_async_copy(
            o_stage.at[0], o_hbm.at[pl.ds(0, _BAND)], o_sem.at[slot]).wait()


@jax.jit
def kernel(q, corpus, A_all, B_all, W_base):
    Bt = jnp.swapaxes(B_all, 1, 2)
    return pl.pallas_call(
        _fused_kernel,
        out_shape=jax.ShapeDtypeStruct((D_MODEL, D_MODEL), jnp.float32),
        in_specs=[
            pl.BlockSpec(memory_space=pltpu.MemorySpace.VMEM),
            pl.BlockSpec(memory_space=pltpu.MemorySpace.HBM),
            pl.BlockSpec(memory_space=pltpu.MemorySpace.HBM),
            pl.BlockSpec(memory_space=pltpu.MemorySpace.HBM),
            pl.BlockSpec(memory_space=pltpu.MemorySpace.HBM),
        ],
        out_specs=pl.BlockSpec(memory_space=pltpu.MemorySpace.HBM),
        scratch_shapes=[
            pltpu.SMEM((TOPK + 1,), jnp.int32),
            pltpu.SMEM((TOPK + 1,), jnp.float32),
            pltpu.VMEM((N_CLUSTERS, D_EMB), jnp.float32),
            pltpu.VMEM((D_MODEL, D_MODEL), jnp.float32),
            pltpu.VMEM((TOPK, R, D_MODEL), jnp.float32),
            pltpu.VMEM((TOPK, R, D_MODEL), jnp.float32),
            pltpu.VMEM((2, _BAND, D_MODEL), jnp.float32),
            pltpu.SemaphoreType.DMA((len(_CHUNKS),)),
            pltpu.SemaphoreType.DMA,
            pltpu.SemaphoreType.DMA,
            pltpu.SemaphoreType.DMA,
            pltpu.SemaphoreType.DMA((2,)),
        ],
    )(q, corpus, W_base, A_all, Bt)


# speedup vs baseline: 30.8958x; 1.0541x over previous
import jax
import jax.numpy as jnp
from jax import lax
from jax.experimental import pallas as pl
from jax.experimental.pallas import tpu as pltpu

N_CLUSTERS = 1000
D_EMB = 1024
D_MODEL = 1024
R = 16
TOPK = 50
BETA = 0.2
TAU = 0.01
SCALING = 2.0

_CHUNKS = (256, 256, 256, 232)


def _fused_kernel(q_ref, corpus_hbm, wb_hbm, a_hbm, bt_hbm, o_ref,
                  idx_smem, val_smem, c_buf, w_buf, a_buf, bt_buf,
                  c_sem, w_sem, a_sem, bt_sem):
    f32 = jnp.float32

    lo = 0
    for ci, sz in enumerate(_CHUNKS):
        pltpu.make_async_copy(
            corpus_hbm.at[pl.ds(lo, sz)], c_buf.at[pl.ds(lo, sz)],
            c_sem.at[ci]).start()
        lo += sz
    pltpu.make_async_copy(wb_hbm, w_buf, w_sem).start()

    q = q_ref[...]
    qn = q / (jnp.sqrt(jnp.sum(q * q)) + 1e-9)
    ones_row = jnp.ones((1, D_EMB), f32)

    def chunk_sims(lo, sz, ci):
        pltpu.make_async_copy(
            corpus_hbm.at[pl.ds(0, sz)], c_buf.at[pl.ds(0, sz)],
            c_sem.at[ci]).wait()
        c = c_buf[pl.ds(lo, sz), :]
        dots = lax.dot_general(
            qn, c, (((1,), (1,)), ((), ())),
            preferred_element_type=f32)
        sq = lax.dot_general(
            ones_row, c * c, (((1,), (1,)), ((), ())),
            preferred_element_type=f32)
        return dots / (jnp.sqrt(sq) + 1e-9) / (BETA * BETA)

    pieces = []
    lo = 0
    for ci, sz in enumerate(_CHUNKS):
        pieces.append(chunk_sims(lo, sz, ci))
        lo += sz
    sim = jnp.concatenate(pieces, axis=1)

    m = jnp.max(sim)
    e = jnp.exp(sim - m)
    probs = e / jnp.sum(e)
    probs = jnp.where(probs >= TAU, probs, 0.0)

    iota = lax.broadcasted_iota(jnp.int32, (1, N_CLUSTERS), 1)
    count = jnp.sum(jnp.where(probs > 0.0, 1, 0))
    nnz = jnp.minimum(count, TOPK)

    def topk_body(k, carry):
        p, total = carry
        v = jnp.max(p)
        i = jnp.min(jnp.where(p == v, iota, N_CLUSTERS))
        i = jnp.minimum(i, N_CLUSTERS - 1)
        idx_smem[k] = i
        val_smem[k] = v
        return jnp.where(iota == i, -1.0, p), total + v

    _, total = lax.fori_loop(0, nnz, topk_body, (probs, jnp.float32(0.0)))
    inv_total = SCALING / (total + 1e-9)

    def start_fetch(k, slot):
        i = idx_smem[k]
        pltpu.make_async_copy(a_hbm.at[i], a_buf.at[slot], a_sem.at[slot]).start()
        pltpu.make_async_copy(bt_hbm.at[i], bt_buf.at[slot], bt_sem.at[slot]).start()

    @pl.when(nnz > 0)
    def _():
        start_fetch(0, 0)

    pltpu.make_async_copy(wb_hbm, w_buf, w_sem).wait()
    out = w_buf[...]

    def merge_body(k, acc):
        slot = lax.rem(k, 2)

        @pl.when(k + 1 < nnz)
        def _():
            start_fetch(k + 1, lax.rem(k + 1, 2))

        pltpu.make_async_copy(a_hbm.at[0], a_buf.at[slot], a_sem.at[slot]).wait()
        pltpu.make_async_copy(bt_hbm.at[0], bt_buf.at[slot], bt_sem.at[slot]).wait()
        w = val_smem[k] * inv_total
        return acc + lax.dot_general(
            bt_buf[slot], a_buf[slot] * w, (((0,), (0,)), ((), ())),
            preferred_element_type=f32)

    o_ref[...] = lax.fori_loop(0, nnz, merge_body, out)


@jax.jit
def kernel(q, corpus, A_all, B_all, W_base):
    Bt = jnp.swapaxes(B_all, 1, 2)
    return pl.pallas_call(
        _fused_kernel,
        out_shape=jax.ShapeDtypeStruct((D_MODEL, D_MODEL), jnp.float32),
        in_specs=[
            pl.BlockSpec(memory_space=pltpu.MemorySpace.VMEM),
            pl.BlockSpec(memory_space=pltpu.MemorySpace.HBM),
            pl.BlockSpec(memory_space=pltpu.MemorySpace.HBM),
            pl.BlockSpec(memory_space=pltpu.MemorySpace.HBM),
            pl.BlockSpec(memory_space=pltpu.MemorySpace.HBM),
        ],
        out_specs=pl.BlockSpec(memory_space=pltpu.MemorySpace.VMEM),
        scratch_shapes=[
            pltpu.SMEM((TOPK + 1,), jnp.int32),
            pltpu.SMEM((TOPK + 1,), jnp.float32),
            pltpu.VMEM((N_CLUSTERS, D_EMB), jnp.float32),
            pltpu.VMEM((D_MODEL, D_MODEL), jnp.float32),
            pltpu.VMEM((2, R, D_MODEL), jnp.float32),
            pltpu.VMEM((2, R, D_MODEL), jnp.float32),
            pltpu.SemaphoreType.DMA((len(_CHUNKS),)),
            pltpu.SemaphoreType.DMA,
            pltpu.SemaphoreType.DMA((2,)),
            pltpu.SemaphoreType.DMA((2,)),
        ],
    )(q, corpus, W_base, A_all, Bt)


# speedup vs baseline: 35.8403x; 1.1600x over previous
import jax
import jax.numpy as jnp
from jax import lax
from jax.experimental import pallas as pl
from jax.experimental.pallas import tpu as pltpu

N_CLUSTERS = 1000
D_EMB = 1024
D_MODEL = 1024
R = 16
TOPK = 50
BETA = 0.2
TAU = 0.01
SCALING = 2.0

_CHUNKS = (256, 256, 256, 232)


def _fused_kernel(q_ref, corpus_hbm, wb_hbm, a_hbm, bt_hbm, o_ref,
                  idx_smem, val_smem, c_buf, a_buf, bt_buf,
                  c_sem, w_sem, a_sem, bt_sem):
    f32 = jnp.float32

    lo = 0
    for ci, sz in enumerate(_CHUNKS):
        pltpu.make_async_copy(
            corpus_hbm.at[pl.ds(lo, sz)], c_buf.at[pl.ds(lo, sz)],
            c_sem.at[ci]).start()
        lo += sz
    pltpu.make_async_copy(wb_hbm, o_ref, w_sem).start()

    q = q_ref[...]
    qn = q / (jnp.sqrt(jnp.sum(q * q)) + 1e-9)
    ones_row = jnp.ones((1, D_EMB), f32)

    def chunk_sims(lo, sz, ci):
        pltpu.make_async_copy(
            corpus_hbm.at[pl.ds(0, sz)], c_buf.at[pl.ds(0, sz)],
            c_sem.at[ci]).wait()
        c = c_buf[pl.ds(lo, sz), :]
        dots = lax.dot_general(
            qn, c, (((1,), (1,)), ((), ())),
            preferred_element_type=f32)
        sq = lax.dot_general(
            ones_row, c * c, (((1,), (1,)), ((), ())),
            preferred_element_type=f32)
        return dots / (jnp.sqrt(sq) + 1e-9) / (BETA * BETA)

    pieces = []
    lo = 0
    for ci, sz in enumerate(_CHUNKS):
        pieces.append(chunk_sims(lo, sz, ci))
        lo += sz
    sim = jnp.concatenate(pieces, axis=1)

    m = jnp.max(sim)
    e = jnp.exp(sim - m)
    probs = e / jnp.sum(e)
    probs = jnp.where(probs >= TAU, probs, 0.0)

    iota = lax.broadcasted_iota(jnp.int32, (1, N_CLUSTERS), 1)
    count = jnp.sum(jnp.where(probs > 0.0, 1, 0))
    nnz = jnp.minimum(count, TOPK)

    def topk_body(k, carry):
        p, total = carry
        v = jnp.max(p)
        i = jnp.min(jnp.where(p == v, iota, N_CLUSTERS))
        i = jnp.minimum(i, N_CLUSTERS - 1)
        idx_smem[k] = i
        val_smem[k] = v
        return jnp.where(iota == i, -1.0, p), total + v

    _, total = lax.fori_loop(0, nnz, topk_body, (probs, jnp.float32(0.0)))
    inv_total = SCALING / (total + 1e-9)

    def start_fetch(k, slot):
        i = idx_smem[k]
        pltpu.make_async_copy(a_hbm.at[i], a_buf.at[slot], a_sem.at[slot]).start()
        pltpu.make_async_copy(bt_hbm.at[i], bt_buf.at[slot], bt_sem.at[slot]).start()

    @pl.when(nnz > 0)
    def _():
        start_fetch(0, 0)

    pltpu.make_async_copy(wb_hbm, o_ref, w_sem).wait()

    def merge_body(k, _):
        slot = lax.rem(k, 2)

        @pl.when(k + 1 < nnz)
        def _():
            start_fetch(k + 1, lax.rem(k + 1, 2))

        pltpu.make_async_copy(a_hbm.at[0], a_buf.at[slot], a_sem.at[slot]).wait()
        pltpu.make_async_copy(bt_hbm.at[0], bt_buf.at[slot], bt_sem.at[slot]).wait()
        w = val_smem[k] * inv_total
        o_ref[...] += lax.dot_general(
            bt_buf[slot], a_buf[slot] * w, (((0,), (0,)), ((), ())),
            preferred_element_type=f32)
        return 0

    lax.fori_loop(0, nnz, merge_body, 0)


@jax.jit
def kernel(q, corpus, A_all, B_all, W_base):
    Bt = jnp.swapaxes(B_all, 1, 2)
    return pl.pallas_call(
        _fused_kernel,
        out_shape=jax.ShapeDtypeStruct((D_MODEL, D_MODEL), jnp.float32),
        in_specs=[
            pl.BlockSpec(memory_space=pltpu.MemorySpace.VMEM),
            pl.BlockSpec(memory_space=pltpu.MemorySpace.HBM),
            pl.BlockSpec(memory_space=pltpu.MemorySpace.HBM),
            pl.BlockSpec(memory_space=pltpu.MemorySpace.HBM),
            pl.BlockSpec(memory_space=pltpu.MemorySpace.HBM),
        ],
        out_specs=pl.BlockSpec(memory_space=pltpu.MemorySpace.VMEM),
        scratch_shapes=[
            pltpu.SMEM((TOPK + 1,), jnp.int32),
            pltpu.SMEM((TOPK + 1,), jnp.float32),
            pltpu.VMEM((N_CLUSTERS, D_EMB), jnp.float32),
            pltpu.VMEM((2, R, D_MODEL), jnp.float32),
            pltpu.VMEM((2, R, D_MODEL), jnp.float32),
            pltpu.SemaphoreType.DMA((len(_CHUNKS),)),
            pltpu.SemaphoreType.DMA,
            pltpu.SemaphoreType.DMA((2,)),
            pltpu.SemaphoreType.DMA((2,)),
        ],
    )(q, corpus, W_base, A_all, Bt)
